# Initial kernel scaffold; baseline (speedup 1.0000x reference)
import jax
import jax.numpy as jnp
from jax.experimental import pallas as pl


def kernel(x, edge_index, edge_attr, params):
    raise NotImplementedError("write your pallas kernel here")


# speedup vs baseline: 13.3938x; 13.3938x over previous
import functools

import jax
import jax.numpy as jnp
from jax import lax
from jax.experimental import pallas as pl
from jax.experimental.pallas import tpu as pltpu
from jax.experimental.pallas import tpu_sc as plsc

NC = 2
NS = 16
NW = NC * NS

EMB = 128
PAD_ROWS = 112
CHUNK = 120


def _edge_kernel(n_nodes, n_pad_rows, e_pad):
  npr = n_nodes + n_pad_rows
  epw = e_pad // NW
  nch = epw // CHUNK
  rpt_p = npr // NS
  mesh = plsc.VectorSubcoreMesh(core_axis_name="c", subcore_axis_name="s",
                                num_cores=NC, num_subcores=NS)

  @functools.partial(
      pl.kernel,
      mesh=mesh,
      out_type=jax.ShapeDtypeStruct((NC, npr, EMB), jnp.float32),
      scratch_types=[
          pltpu.VMEM((CHUNK,), jnp.int32),
          pltpu.VMEM((CHUNK,), jnp.int32),
          pltpu.VMEM((CHUNK, EMB), jnp.float32),
          pltpu.VMEM_SHARED((npr, EMB), jnp.float32),
          pltpu.SemaphoreType.DMA,
      ],
  )
  def k(src_hbm, dst_hbm, g_hbm, zeros_hbm, out_hbm, sidx_v, didx_v, rows_v,
        acc_sh, sem):
    c = lax.axis_index("c")
    s = lax.axis_index("s")
    wid = s * NC + c
    pltpu.sync_copy(zeros_hbm.at[pl.ds(s * rpt_p, rpt_p)],
                    acc_sh.at[pl.ds(s * rpt_p, rpt_p)])
    plsc.subcore_barrier()

    def body(j, carry):
      base = wid * epw + j * CHUNK
      pltpu.sync_copy(src_hbm.at[pl.ds(base, CHUNK)], sidx_v)
      pltpu.sync_copy(dst_hbm.at[pl.ds(base, CHUNK)], didx_v)
      pltpu.async_copy(g_hbm.at[sidx_v], rows_v, sem).wait()
      pltpu.sync_copy(rows_v, acc_sh.at[didx_v], add=True)
      return carry

    lax.fori_loop(0, nch, body, 0)
    plsc.subcore_barrier()
    pltpu.sync_copy(acc_sh.at[pl.ds(s * rpt_p, rpt_p)],
                    out_hbm.at[c, pl.ds(s * rpt_p, rpt_p)])

  return k


def _deg_kernel(deg_pad, e_pad):
  epw = e_pad // NW
  nch = epw // CHUNK
  ept = deg_pad // NS
  mesh = plsc.VectorSubcoreMesh(core_axis_name="c", subcore_axis_name="s",
                                num_cores=NC, num_subcores=NS)

  @functools.partial(
      pl.kernel,
      mesh=mesh,
      out_type=jax.ShapeDtypeStruct((NC, deg_pad), jnp.float32),
      scratch_types=[
          pltpu.VMEM((CHUNK,), jnp.int32),
          pltpu.VMEM((CHUNK,), jnp.float32),
          pltpu.VMEM_SHARED((deg_pad,), jnp.float32),
      ],
  )
  def k(dst_hbm, zeros_hbm, ones_hbm, out_hbm, didx_v, ones_v, acc_sh):
    c = lax.axis_index("c")
    s = lax.axis_index("s")
    wid = s * NC + c
    pltpu.sync_copy(zeros_hbm.at[pl.ds(s * ept, ept)],
                    acc_sh.at[pl.ds(s * ept, ept)])
    pltpu.sync_copy(ones_hbm, ones_v)
    plsc.subcore_barrier()

    def body(j, carry):
      base = wid * epw + j * CHUNK
      pltpu.sync_copy(dst_hbm.at[pl.ds(base, CHUNK)], didx_v)
      pltpu.sync_copy(ones_v, acc_sh.at[didx_v], add=True)
      return carry

    lax.fori_loop(0, nch, body, 0)
    plsc.subcore_barrier()
    pltpu.sync_copy(acc_sh.at[pl.ds(s * ept, ept)],
                    out_hbm.at[c, pl.ds(s * ept, ept)])

  return k


def _dense0_body(x_ref, w_ref, b_ref, degp_ref, g_ref, dinv_ref, *, n, npr):
  deg = degp_ref[0, :n] + degp_ref[1, :n]
  dinv = lax.rsqrt(jnp.maximum(deg, 1e-12))[:, None]
  h = jnp.dot(x_ref[...], w_ref[...], preferred_element_type=jnp.float32)
  g = (h + b_ref[...][None, :]) * dinv
  g_ref[:n, :] = g
  g_ref[n:, :] = jnp.zeros((npr - n, EMB), jnp.float32)
  dinv_ref[...] = dinv


def _bn(y, gamma, beta):
  mean = jnp.mean(y, axis=0, keepdims=True)
  d = y - mean
  var = jnp.mean(d * d, axis=0, keepdims=True)
  return d * lax.rsqrt(var + 1e-5) * gamma[None, :] + beta[None, :]


def _layer_body(sp_ref, dinv_ref, w_ref, b_ref, gm_ref, bt_ref, g_ref, *, n,
                npr):
  agg = (sp_ref[0, :n] + sp_ref[1, :n]) * dinv_ref[...]
  y = jnp.dot(agg, w_ref[...], preferred_element_type=jnp.float32)
  h = _bn(y + b_ref[...][None, :], gm_ref[...], bt_ref[...])
  h = jnp.maximum(h, 0.0)
  g_ref[:n, :] = h * dinv_ref[...]
  g_ref[n:, :] = jnp.zeros((npr - n, EMB), jnp.float32)


def _final_body(sp_ref, dinv_ref, w_ref, b_ref, gm_ref, bt_ref, fc0w_ref,
                fc0b_ref, fc1w_ref, fc1b_ref, out_ref, *, n):
  agg = (sp_ref[0, :n] + sp_ref[1, :n]) * dinv_ref[...]
  y = jnp.dot(agg, w_ref[...], preferred_element_type=jnp.float32)
  h = _bn(y + b_ref[...][None, :], gm_ref[...], bt_ref[...])
  z = jnp.dot(h, fc0w_ref[...], preferred_element_type=jnp.float32)
  z = jnp.maximum(z + fc0b_ref[...][None, :], 0.0)
  o = jnp.dot(z, fc1w_ref[...], preferred_element_type=jnp.float32)
  out_ref[...] = o + fc1b_ref[...][None, :]


def kernel(x, edge_index, edge_attr, params):
  n = x.shape[0]
  e = edge_index.shape[1]
  npr = n + PAD_ROWS
  assert npr % NS == 0 and n % NS == 0

  e_f = e + n
  nch = -(-e_f // (NW * CHUNK))
  e_pad = nch * NW * CHUNK
  n_dummy = e_pad - e_f

  src = edge_index[0].astype(jnp.int32)
  dst = edge_index[1].astype(jnp.int32)
  loop = jnp.arange(n, dtype=jnp.int32)
  padidx = n + (jnp.arange(n_dummy, dtype=jnp.int32) % PAD_ROWS)
  src_f = jnp.concatenate([src, loop, padidx])
  dst_f = jnp.concatenate([dst, loop, padidx])

  deg_pad = -(-(n + PAD_ROWS) // (NS * 128)) * NS * 128
  zeros2d = jnp.zeros((npr, EMB), jnp.float32)
  zeros1d = jnp.zeros((deg_pad,), jnp.float32)
  ones_c = jnp.ones((CHUNK,), jnp.float32)

  edge_k = _edge_kernel(n, PAD_ROWS, e_pad)
  deg_k = _deg_kernel(deg_pad, e_pad)

  degp = deg_k(dst_f, zeros1d, ones_c)

  g, dinv = pl.pallas_call(
      functools.partial(_dense0_body, n=n, npr=npr),
      out_shape=(jax.ShapeDtypeStruct((npr, EMB), jnp.float32),
                 jax.ShapeDtypeStruct((n, 1), jnp.float32)),
  )(x, params['lin_x_W'], params['lin_x_b'], degp)

  for l in range(2):
    sp = edge_k(src_f, dst_f, g, zeros2d)
    g = pl.pallas_call(
        functools.partial(_layer_body, n=n, npr=npr),
        out_shape=jax.ShapeDtypeStruct((npr, EMB), jnp.float32),
    )(sp, dinv, params['W%d' % l], params['b%d' % l], params['gamma%d' % l],
      params['beta%d' % l])

  sp = edge_k(src_f, dst_f, g, zeros2d)
  out = pl.pallas_call(
      functools.partial(_final_body, n=n),
      out_shape=jax.ShapeDtypeStruct((n, 1), jnp.float32),
  )(sp, dinv, params['W2'], params['b2'], params['gamma2'], params['beta2'],
    params['fc0_W'], params['fc0_b'], params['fc1_W'], params['fc1_b'])
  return out


# speedup vs baseline: 27.2517x; 2.0347x over previous
import functools

import jax
import jax.numpy as jnp
from jax import lax
from jax.experimental import pallas as pl
from jax.experimental.pallas import tpu as pltpu
from jax.experimental.pallas import tpu_sc as plsc

NC = 2
NS = 16
NW = NC * NS

EMB = 128
PAD_ROWS = 112
CHUNK = 96
NBUF = 2
NBI = 4


def _edge_kernel(n_nodes, n_pad_rows, e_pad):
  npr = n_nodes + n_pad_rows
  epw = e_pad // NW
  nch = epw // CHUNK
  rpt_p = npr // NS
  mesh = plsc.VectorSubcoreMesh(core_axis_name="c", subcore_axis_name="s",
                                num_cores=NC, num_subcores=NS)

  assert nch % NBI == 0 and NBI % NBUF == 0

  @functools.partial(
      pl.kernel,
      mesh=mesh,
      out_type=jax.ShapeDtypeStruct((NC, npr, EMB), jnp.float32),
      scratch_types=[
          pltpu.VMEM((nch, CHUNK), jnp.int32),
          [pltpu.VMEM((CHUNK,), jnp.int32)] * NBI,
          [pltpu.VMEM((CHUNK, EMB), jnp.float32)] * NBUF,
          pltpu.VMEM_SHARED((npr, EMB), jnp.float32),
          [pltpu.SemaphoreType.DMA] * NBUF,
          [pltpu.SemaphoreType.DMA] * NBI,
      ],
  )
  def k(src_hbm, dst_hbm, g_hbm, zeros_hbm, out_hbm, sidx_v, didx_b, rows_v,
        acc_sh, sem_g, sem_d):
    c = lax.axis_index("c")
    s = lax.axis_index("s")
    wid = s * NC + c
    pltpu.sync_copy(src_hbm.at[wid], sidx_v)
    pltpu.sync_copy(zeros_hbm.at[pl.ds(s * rpt_p, rpt_p)],
                    acc_sh.at[pl.ds(s * rpt_p, rpt_p)])
    plsc.subcore_barrier()

    def dst_slice(j):
      return dst_hbm.at[pl.ds(wid * epw + j * CHUNK, CHUNK)]

    for bd in range(NBI):
      pltpu.async_copy(dst_slice(bd), didx_b[bd], sem_d[bd])
    for b in range(NBUF):
      pltpu.async_copy(g_hbm.at[sidx_v.at[b]], rows_v[b], sem_g[b])

    def body(jj, carry):
      for u in range(NBI):
        j = jj * NBI + u
        b = u % NBUF
        pltpu.make_async_copy(g_hbm.at[sidx_v.at[j]], rows_v[b],
                              sem_g[b]).wait()
        pltpu.make_async_copy(dst_slice(j), didx_b[u], sem_d[u]).wait()
        pltpu.sync_copy(rows_v[b], acc_sh.at[didx_b[u]], add=True)

        @pl.when(j + NBI < nch)
        def _():
          pltpu.async_copy(dst_slice(j + NBI), didx_b[u], sem_d[u])

        @pl.when(j + NBUF < nch)
        def _():
          pltpu.async_copy(g_hbm.at[sidx_v.at[j + NBUF]], rows_v[b],
                           sem_g[b])

      return carry

    lax.fori_loop(0, nch // NBI, body, 0)
    plsc.subcore_barrier()
    pltpu.sync_copy(acc_sh.at[pl.ds(s * rpt_p, rpt_p)],
                    out_hbm.at[c, pl.ds(s * rpt_p, rpt_p)])

  return k


def _deg_kernel(deg_pad, e_pad):
  epw = e_pad // NW
  nch = epw // CHUNK
  ept = deg_pad // NS
  mesh = plsc.VectorSubcoreMesh(core_axis_name="c", subcore_axis_name="s",
                                num_cores=NC, num_subcores=NS)

  assert nch % NBUF == 0

  @functools.partial(
      pl.kernel,
      mesh=mesh,
      out_type=jax.ShapeDtypeStruct((NC, deg_pad), jnp.float32),
      scratch_types=[
          pltpu.VMEM((nch, CHUNK), jnp.int32),
          pltpu.VMEM((CHUNK,), jnp.float32),
          pltpu.VMEM_SHARED((deg_pad,), jnp.float32),
          [pltpu.SemaphoreType.DMA] * NBUF,
      ],
  )
  def k(dst_hbm, zeros_hbm, ones_hbm, out_hbm, didx_v, ones_v, acc_sh, sems):
    c = lax.axis_index("c")
    s = lax.axis_index("s")
    wid = s * NC + c
    pltpu.sync_copy(dst_hbm.at[wid], didx_v)
    pltpu.sync_copy(zeros_hbm.at[pl.ds(s * ept, ept)],
                    acc_sh.at[pl.ds(s * ept, ept)])
    pltpu.sync_copy(ones_hbm, ones_v)
    plsc.subcore_barrier()

    for b in range(NBUF):
      pltpu.async_copy(ones_v, acc_sh.at[didx_v.at[b]], sems[b], add=True)

    def body(jj, carry):
      for b in range(NBUF):
        j = jj * NBUF + b
        pltpu.make_async_copy(ones_v, acc_sh.at[didx_v.at[j]],
                              sems[b]).wait()

        @pl.when(jj * NBUF + NBUF + b < nch)
        def _():
          pltpu.async_copy(ones_v, acc_sh.at[didx_v.at[jj * NBUF + NBUF + b]],
                           sems[b], add=True)

      return carry

    lax.fori_loop(0, nch // NBUF, body, 0)
    plsc.subcore_barrier()
    pltpu.sync_copy(acc_sh.at[pl.ds(s * ept, ept)],
                    out_hbm.at[c, pl.ds(s * ept, ept)])

  return k


def _dense0_body(x_ref, w_ref, b_ref, degp_ref, g_ref, dinv_ref, *, n, npr):
  deg = degp_ref[0, :n] + degp_ref[1, :n]
  dinv = lax.rsqrt(jnp.maximum(deg, 1e-12))[:, None]
  h = jnp.dot(x_ref[...], w_ref[...], preferred_element_type=jnp.float32)
  g = (h + b_ref[...][None, :]) * dinv
  g_ref[:n, :] = g
  g_ref[n:, :] = jnp.zeros((npr - n, EMB), jnp.float32)
  dinv_ref[...] = dinv


def _bn(y, gamma, beta):
  mean = jnp.mean(y, axis=0, keepdims=True)
  d = y - mean
  var = jnp.mean(d * d, axis=0, keepdims=True)
  return d * lax.rsqrt(var + 1e-5) * gamma[None, :] + beta[None, :]


def _layer_body(sp_ref, dinv_ref, w_ref, b_ref, gm_ref, bt_ref, g_ref, *, n,
                npr):
  agg = (sp_ref[0, :n] + sp_ref[1, :n]) * dinv_ref[...]
  y = jnp.dot(agg, w_ref[...], preferred_element_type=jnp.float32)
  h = _bn(y + b_ref[...][None, :], gm_ref[...], bt_ref[...])
  h = jnp.maximum(h, 0.0)
  g_ref[:n, :] = h * dinv_ref[...]
  g_ref[n:, :] = jnp.zeros((npr - n, EMB), jnp.float32)


def _final_body(sp_ref, dinv_ref, w_ref, b_ref, gm_ref, bt_ref, fc0w_ref,
                fc0b_ref, fc1w_ref, fc1b_ref, out_ref, *, n):
  agg = (sp_ref[0, :n] + sp_ref[1, :n]) * dinv_ref[...]
  y = jnp.dot(agg, w_ref[...], preferred_element_type=jnp.float32)
  h = _bn(y + b_ref[...][None, :], gm_ref[...], bt_ref[...])
  z = jnp.dot(h, fc0w_ref[...], preferred_element_type=jnp.float32)
  z = jnp.maximum(z + fc0b_ref[...][None, :], 0.0)
  o = jnp.dot(z, fc1w_ref[...], preferred_element_type=jnp.float32)
  out_ref[...] = o + fc1b_ref[...][None, :]


def kernel(x, edge_index, edge_attr, params):
  n = x.shape[0]
  e = edge_index.shape[1]
  npr = n + PAD_ROWS
  assert npr % NS == 0 and n % NS == 0

  e_f = e + n
  nch = -(-e_f // (NW * CHUNK * NBI)) * NBI
  e_pad = nch * NW * CHUNK
  n_dummy = e_pad - e_f

  src = edge_index[0].astype(jnp.int32)
  dst = edge_index[1].astype(jnp.int32)
  loop = jnp.arange(n, dtype=jnp.int32)
  padidx = n + (jnp.arange(n_dummy, dtype=jnp.int32) % PAD_ROWS)
  src_f = jnp.concatenate([src, loop, padidx]).reshape(NW, nch, CHUNK)
  dst_1d = jnp.concatenate([dst, loop, padidx])
  dst_f = dst_1d.reshape(NW, nch, CHUNK)

  deg_pad = -(-(n + PAD_ROWS) // (NS * 128)) * NS * 128
  zeros2d = jnp.zeros((npr, EMB), jnp.float32)
  zeros1d = jnp.zeros((deg_pad,), jnp.float32)
  ones_c = jnp.ones((CHUNK,), jnp.float32)

  edge_k = _edge_kernel(n, PAD_ROWS, e_pad)
  deg_k = _deg_kernel(deg_pad, e_pad)

  degp = deg_k(dst_f, zeros1d, ones_c)

  g, dinv = pl.pallas_call(
      functools.partial(_dense0_body, n=n, npr=npr),
      out_shape=(jax.ShapeDtypeStruct((npr, EMB), jnp.float32),
                 jax.ShapeDtypeStruct((n, 1), jnp.float32)),
  )(x, params['lin_x_W'], params['lin_x_b'], degp)

  for l in range(2):
    sp = edge_k(src_f, dst_1d, g, zeros2d)
    g = pl.pallas_call(
        functools.partial(_layer_body, n=n, npr=npr),
        out_shape=jax.ShapeDtypeStruct((npr, EMB), jnp.float32),
    )(sp, dinv, params['W%d' % l], params['b%d' % l], params['gamma%d' % l],
      params['beta%d' % l])

  sp = edge_k(src_f, dst_1d, g, zeros2d)
  out = pl.pallas_call(
      functools.partial(_final_body, n=n),
      out_shape=jax.ShapeDtypeStruct((n, 1), jnp.float32),
  )(sp, dinv, params['W2'], params['b2'], params['gamma2'], params['beta2'],
    params['fc0_W'], params['fc0_b'], params['fc1_W'], params['fc1_b'])
  return out


# speedup vs baseline: 31.1921x; 1.1446x over previous
import functools

import jax
import jax.numpy as jnp
from jax import lax
from jax.experimental import pallas as pl
from jax.experimental.pallas import tpu as pltpu
from jax.experimental.pallas import tpu_sc as plsc

NC = 2
NS = 16
NW = NC * NS

EMB = 128
PAD_ROWS = 112
CHUNK = 96
NBUF = 3
NBI = 6


def _edge_kernel(n_nodes, n_pad_rows, e_pad):
  npr = n_nodes + n_pad_rows
  epw = e_pad // NW
  nch = epw // CHUNK
  rpt_p = npr // NS
  mesh = plsc.VectorSubcoreMesh(core_axis_name="c", subcore_axis_name="s",
                                num_cores=NC, num_subcores=NS)

  assert nch % NBI == 0 and NBI % NBUF == 0

  @functools.partial(
      pl.kernel,
      mesh=mesh,
      out_type=jax.ShapeDtypeStruct((NC, npr, EMB), jnp.float32),
      scratch_types=[
          [pltpu.VMEM((CHUNK,), jnp.int32)] * NBI,
          [pltpu.VMEM((CHUNK,), jnp.int32)] * NBI,
          [pltpu.VMEM((CHUNK, EMB), jnp.float32)] * NBUF,
          pltpu.VMEM_SHARED((npr, EMB), jnp.float32),
          [pltpu.SemaphoreType.DMA] * NBUF,
          [pltpu.SemaphoreType.DMA] * NBI,
          [pltpu.SemaphoreType.DMA] * NBI,
      ],
  )
  def k(src_hbm, dst_hbm, g_hbm, zeros_hbm, out_hbm, sidx_b, didx_b, rows_v,
        acc_sh, sem_g, sem_d, sem_s):
    c = lax.axis_index("c")
    s = lax.axis_index("s")
    wid = s * NC + c
    pltpu.sync_copy(zeros_hbm.at[pl.ds(s * rpt_p, rpt_p)],
                    acc_sh.at[pl.ds(s * rpt_p, rpt_p)])
    plsc.subcore_barrier()

    def src_slice(j):
      return src_hbm.at[pl.ds(wid * epw + j * CHUNK, CHUNK)]

    def dst_slice(j):
      return dst_hbm.at[pl.ds(wid * epw + j * CHUNK, CHUNK)]

    for u in range(NBI):
      pltpu.async_copy(src_slice(u), sidx_b[u], sem_s[u])
      pltpu.async_copy(dst_slice(u), didx_b[u], sem_d[u])
    for b in range(NBUF):
      pltpu.make_async_copy(src_slice(b), sidx_b[b], sem_s[b]).wait()
      pltpu.async_copy(g_hbm.at[sidx_b[b]], rows_v[b], sem_g[b])

    def body(jj, carry):
      for u in range(NBI):
        j = jj * NBI + u
        b = u % NBUF
        u3 = (u + NBUF) % NBI
        pltpu.make_async_copy(g_hbm.at[sidx_b[b]], rows_v[b],
                              sem_g[b]).wait()
        pltpu.make_async_copy(dst_slice(j), didx_b[u], sem_d[u]).wait()
        pltpu.sync_copy(rows_v[b], acc_sh.at[didx_b[u]], add=True)

        @pl.when(j + NBI < nch)
        def _():
          pltpu.async_copy(src_slice(j + NBI), sidx_b[u], sem_s[u])
          pltpu.async_copy(dst_slice(j + NBI), didx_b[u], sem_d[u])

        @pl.when(j + NBUF < nch)
        def _():
          pltpu.make_async_copy(src_slice(j + NBUF), sidx_b[u3],
                                sem_s[u3]).wait()
          pltpu.async_copy(g_hbm.at[sidx_b[u3]], rows_v[b], sem_g[b])

      return carry

    lax.fori_loop(0, nch // NBI, body, 0)
    plsc.subcore_barrier()
    pltpu.sync_copy(acc_sh.at[pl.ds(s * rpt_p, rpt_p)],
                    out_hbm.at[c, pl.ds(s * rpt_p, rpt_p)])

  return k


def _deg_kernel(deg_pad, e_pad):
  epw = e_pad // NW
  nch = epw // CHUNK
  ept = deg_pad // NS
  mesh = plsc.VectorSubcoreMesh(core_axis_name="c", subcore_axis_name="s",
                                num_cores=NC, num_subcores=NS)

  assert nch % NBUF == 0

  @functools.partial(
      pl.kernel,
      mesh=mesh,
      out_type=jax.ShapeDtypeStruct((NC, deg_pad), jnp.float32),
      scratch_types=[
          pltpu.VMEM((nch, CHUNK), jnp.int32),
          pltpu.VMEM((CHUNK,), jnp.float32),
          pltpu.VMEM_SHARED((deg_pad,), jnp.float32),
          [pltpu.SemaphoreType.DMA] * NBUF,
      ],
  )
  def k(dst_hbm, zeros_hbm, ones_hbm, out_hbm, didx_v, ones_v, acc_sh, sems):
    c = lax.axis_index("c")
    s = lax.axis_index("s")
    wid = s * NC + c
    pltpu.sync_copy(dst_hbm.at[wid], didx_v)
    pltpu.sync_copy(zeros_hbm.at[pl.ds(s * ept, ept)],
                    acc_sh.at[pl.ds(s * ept, ept)])
    pltpu.sync_copy(ones_hbm, ones_v)
    plsc.subcore_barrier()

    for b in range(NBUF):
      pltpu.async_copy(ones_v, acc_sh.at[didx_v.at[b]], sems[b], add=True)

    def body(jj, carry):
      for b in range(NBUF):
        j = jj * NBUF + b
        pltpu.make_async_copy(ones_v, acc_sh.at[didx_v.at[j]],
                              sems[b]).wait()

        @pl.when(jj * NBUF + NBUF + b < nch)
        def _():
          pltpu.async_copy(ones_v, acc_sh.at[didx_v.at[jj * NBUF + NBUF + b]],
                           sems[b], add=True)

      return carry

    lax.fori_loop(0, nch // NBUF, body, 0)
    plsc.subcore_barrier()
    pltpu.sync_copy(acc_sh.at[pl.ds(s * ept, ept)],
                    out_hbm.at[c, pl.ds(s * ept, ept)])

  return k


def _dense0_body(x_ref, w_ref, b_ref, degp_ref, g_ref, dinv_ref, *, n, npr):
  deg = degp_ref[0, :n] + degp_ref[1, :n]
  dinv = lax.rsqrt(jnp.maximum(deg, 1e-12))[:, None]
  h = jnp.dot(x_ref[...], w_ref[...], preferred_element_type=jnp.float32)
  g = (h + b_ref[...][None, :]) * dinv
  g_ref[:n, :] = g
  g_ref[n:, :] = jnp.zeros((npr - n, EMB), jnp.float32)
  dinv_ref[...] = dinv


def _bn(y, gamma, beta):
  mean = jnp.mean(y, axis=0, keepdims=True)
  d = y - mean
  var = jnp.mean(d * d, axis=0, keepdims=True)
  return d * lax.rsqrt(var + 1e-5) * gamma[None, :] + beta[None, :]


def _layer_body(sp_ref, dinv_ref, w_ref, b_ref, gm_ref, bt_ref, g_ref, *, n,
                npr):
  agg = (sp_ref[0, :n] + sp_ref[1, :n]) * dinv_ref[...]
  y = jnp.dot(agg, w_ref[...], preferred_element_type=jnp.float32)
  h = _bn(y + b_ref[...][None, :], gm_ref[...], bt_ref[...])
  h = jnp.maximum(h, 0.0)
  g_ref[:n, :] = h * dinv_ref[...]
  g_ref[n:, :] = jnp.zeros((npr - n, EMB), jnp.float32)


def _final_body(sp_ref, dinv_ref, w_ref, b_ref, gm_ref, bt_ref, fc0w_ref,
                fc0b_ref, fc1w_ref, fc1b_ref, out_ref, *, n):
  agg = (sp_ref[0, :n] + sp_ref[1, :n]) * dinv_ref[...]
  y = jnp.dot(agg, w_ref[...], preferred_element_type=jnp.float32)
  h = _bn(y + b_ref[...][None, :], gm_ref[...], bt_ref[...])
  z = jnp.dot(h, fc0w_ref[...], preferred_element_type=jnp.float32)
  z = jnp.maximum(z + fc0b_ref[...][None, :], 0.0)
  o = jnp.dot(z, fc1w_ref[...], preferred_element_type=jnp.float32)
  out_ref[...] = o + fc1b_ref[...][None, :]


def kernel(x, edge_index, edge_attr, params):
  n = x.shape[0]
  e = edge_index.shape[1]
  npr = n + PAD_ROWS
  assert npr % NS == 0 and n % NS == 0

  e_f = e + n
  nch = -(-e_f // (NW * CHUNK * NBI)) * NBI
  e_pad = nch * NW * CHUNK
  n_dummy = e_pad - e_f

  src = edge_index[0].astype(jnp.int32)
  dst = edge_index[1].astype(jnp.int32)
  loop = jnp.arange(n, dtype=jnp.int32)
  padidx = n + (jnp.arange(n_dummy, dtype=jnp.int32) % PAD_ROWS)
  src_1d = jnp.concatenate([src, loop, padidx])
  dst_1d = jnp.concatenate([dst, loop, padidx])
  dst_f = dst_1d.reshape(NW, nch, CHUNK)

  deg_pad = -(-(n + PAD_ROWS) // (NS * 128)) * NS * 128
  zeros2d = jnp.zeros((npr, EMB), jnp.float32)
  zeros1d = jnp.zeros((deg_pad,), jnp.float32)
  ones_c = jnp.ones((CHUNK,), jnp.float32)

  edge_k = _edge_kernel(n, PAD_ROWS, e_pad)
  deg_k = _deg_kernel(deg_pad, e_pad)

  degp = deg_k(dst_f, zeros1d, ones_c)

  g, dinv = pl.pallas_call(
      functools.partial(_dense0_body, n=n, npr=npr),
      out_shape=(jax.ShapeDtypeStruct((npr, EMB), jnp.float32),
                 jax.ShapeDtypeStruct((n, 1), jnp.float32)),
  )(x, params['lin_x_W'], params['lin_x_b'], degp)

  for l in range(2):
    sp = edge_k(src_1d, dst_1d, g, zeros2d)
    g = pl.pallas_call(
        functools.partial(_layer_body, n=n, npr=npr),
        out_shape=jax.ShapeDtypeStruct((npr, EMB), jnp.float32),
    )(sp, dinv, params['W%d' % l], params['b%d' % l], params['gamma%d' % l],
      params['beta%d' % l])

  sp = edge_k(src_1d, dst_1d, g, zeros2d)
  out = pl.pallas_call(
      functools.partial(_final_body, n=n),
      out_shape=jax.ShapeDtypeStruct((n, 1), jnp.float32),
  )(sp, dinv, params['W2'], params['b2'], params['gamma2'], params['beta2'],
    params['fc0_W'], params['fc0_b'], params['fc1_W'], params['fc1_b'])
  return out


# speedup vs baseline: 31.7024x; 1.0164x over previous
import functools

import jax
import jax.numpy as jnp
from jax import lax
from jax.experimental import pallas as pl
from jax.experimental.pallas import tpu as pltpu
from jax.experimental.pallas import tpu_sc as plsc

NC = 2
NS = 16
NW = NC * NS

EMB = 128
PAD_ROWS = 112
CHUNK = 96
NBUF = 3
NBI = 6


def _edge_kernel(n_nodes, n_pad_rows, e_pad):
  npr = n_nodes + n_pad_rows
  epw = e_pad // NW
  nch = epw // CHUNK
  rpt_p = npr // NS
  mesh = plsc.VectorSubcoreMesh(core_axis_name="c", subcore_axis_name="s",
                                num_cores=NC, num_subcores=NS)

  assert nch % NBI == 0 and NBI % NBUF == 0

  @functools.partial(
      pl.kernel,
      mesh=mesh,
      out_type=jax.ShapeDtypeStruct((NC, npr, EMB), jnp.float32),
      scratch_types=[
          [pltpu.VMEM((CHUNK,), jnp.int32)] * NBI,
          [pltpu.VMEM((CHUNK,), jnp.int32)] * NBI,
          [pltpu.VMEM((CHUNK, EMB), jnp.float32)] * NBUF,
          pltpu.VMEM_SHARED((npr, EMB), jnp.float32),
          [pltpu.SemaphoreType.DMA] * NBUF,
          [pltpu.SemaphoreType.DMA] * NBI,
          [pltpu.SemaphoreType.DMA] * NBI,
      ],
  )
  def k(src_hbm, dst_hbm, g_hbm, zeros_hbm, out_hbm, sidx_b, didx_b, rows_v,
        acc_sh, sem_g, sem_d, sem_s):
    c = lax.axis_index("c")
    s = lax.axis_index("s")
    wid = s * NC + c

    def src_slice(j):
      return src_hbm.at[pl.ds(wid * epw + j * CHUNK, CHUNK)]

    def dst_slice(j):
      return dst_hbm.at[pl.ds(wid * epw + j * CHUNK, CHUNK)]

    for u in range(NBI):
      pltpu.async_copy(src_slice(u), sidx_b[u], sem_s[u])
      pltpu.async_copy(dst_slice(u), didx_b[u], sem_d[u])
    for b in range(NBUF):
      pltpu.make_async_copy(src_slice(b), sidx_b[b], sem_s[b]).wait()
      pltpu.async_copy(g_hbm.at[sidx_b[b]], rows_v[b], sem_g[b])
    pltpu.sync_copy(zeros_hbm.at[pl.ds(s * rpt_p, rpt_p)],
                    acc_sh.at[pl.ds(s * rpt_p, rpt_p)])
    plsc.subcore_barrier()

    def body(jj, carry):
      for u in range(NBI):
        j = jj * NBI + u
        b = u % NBUF
        u3 = (u + NBUF) % NBI
        pltpu.make_async_copy(g_hbm.at[sidx_b[b]], rows_v[b],
                              sem_g[b]).wait()
        pltpu.make_async_copy(dst_slice(j), didx_b[u], sem_d[u]).wait()
        pltpu.sync_copy(rows_v[b], acc_sh.at[didx_b[u]], add=True)

        @pl.when(j + NBI < nch)
        def _():
          pltpu.async_copy(src_slice(j + NBI), sidx_b[u], sem_s[u])
          pltpu.async_copy(dst_slice(j + NBI), didx_b[u], sem_d[u])

        @pl.when(j + NBUF < nch)
        def _():
          pltpu.make_async_copy(src_slice(j + NBUF), sidx_b[u3],
                                sem_s[u3]).wait()
          pltpu.async_copy(g_hbm.at[sidx_b[u3]], rows_v[b], sem_g[b])

      return carry

    lax.fori_loop(0, nch // NBI, body, 0)
    plsc.subcore_barrier()
    pltpu.sync_copy(acc_sh.at[pl.ds(s * rpt_p, rpt_p)],
                    out_hbm.at[c, pl.ds(s * rpt_p, rpt_p)])

  return k


def _deg_kernel(deg_pad, e_pad):
  epw = e_pad // NW
  nch = epw // CHUNK
  ept = deg_pad // NS
  mesh = plsc.VectorSubcoreMesh(core_axis_name="c", subcore_axis_name="s",
                                num_cores=NC, num_subcores=NS)

  assert nch % NBUF == 0

  @functools.partial(
      pl.kernel,
      mesh=mesh,
      out_type=jax.ShapeDtypeStruct((NC, deg_pad), jnp.float32),
      scratch_types=[
          pltpu.VMEM((nch, CHUNK), jnp.int32),
          pltpu.VMEM((CHUNK,), jnp.float32),
          pltpu.VMEM_SHARED((deg_pad,), jnp.float32),
          [pltpu.SemaphoreType.DMA] * NBUF,
      ],
  )
  def k(dst_hbm, zeros_hbm, ones_hbm, out_hbm, didx_v, ones_v, acc_sh, sems):
    c = lax.axis_index("c")
    s = lax.axis_index("s")
    wid = s * NC + c
    pltpu.sync_copy(ones_hbm, ones_v)
    pltpu.sync_copy(dst_hbm.at[wid], didx_v)
    pltpu.sync_copy(zeros_hbm.at[pl.ds(s * ept, ept)],
                    acc_sh.at[pl.ds(s * ept, ept)])
    plsc.subcore_barrier()

    for b in range(NBUF):
      pltpu.async_copy(ones_v, acc_sh.at[didx_v.at[b]], sems[b], add=True)

    def body(jj, carry):
      for b in range(NBUF):
        j = jj * NBUF + b
        pltpu.make_async_copy(ones_v, acc_sh.at[didx_v.at[j]],
                              sems[b]).wait()

        @pl.when(jj * NBUF + NBUF + b < nch)
        def _():
          pltpu.async_copy(ones_v, acc_sh.at[didx_v.at[jj * NBUF + NBUF + b]],
                           sems[b], add=True)

      return carry

    lax.fori_loop(0, nch // NBUF, body, 0)
    plsc.subcore_barrier()
    pltpu.sync_copy(acc_sh.at[pl.ds(s * ept, ept)],
                    out_hbm.at[c, pl.ds(s * ept, ept)])

  return k


def _dense0_body(x_ref, w_ref, b_ref, degp_ref, g_ref, dinv_ref, *, n, npr):
  deg = degp_ref[0, :n] + degp_ref[1, :n]
  dinv = lax.rsqrt(jnp.maximum(deg, 1e-12))[:, None]
  h = jnp.dot(x_ref[...], w_ref[...], preferred_element_type=jnp.float32)
  g = (h + b_ref[...][None, :]) * dinv
  g_ref[:n, :] = g
  g_ref[n:, :] = jnp.zeros((npr - n, EMB), jnp.float32)
  dinv_ref[...] = dinv


def _bn(y, gamma, beta):
  mean = jnp.mean(y, axis=0, keepdims=True)
  d = y - mean
  var = jnp.mean(d * d, axis=0, keepdims=True)
  return d * lax.rsqrt(var + 1e-5) * gamma[None, :] + beta[None, :]


def _layer_body(sp_ref, dinv_ref, w_ref, b_ref, gm_ref, bt_ref, g_ref, *, n,
                npr):
  agg = (sp_ref[0, :n] + sp_ref[1, :n]) * dinv_ref[...]
  y = jnp.dot(agg, w_ref[...], preferred_element_type=jnp.float32)
  h = _bn(y + b_ref[...][None, :], gm_ref[...], bt_ref[...])
  h = jnp.maximum(h, 0.0)
  g_ref[:n, :] = h * dinv_ref[...]
  g_ref[n:, :] = jnp.zeros((npr - n, EMB), jnp.float32)


def _final_body(sp_ref, dinv_ref, w_ref, b_ref, gm_ref, bt_ref, fc0w_ref,
                fc0b_ref, fc1w_ref, fc1b_ref, out_ref, *, n):
  agg = (sp_ref[0, :n] + sp_ref[1, :n]) * dinv_ref[...]
  y = jnp.dot(agg, w_ref[...], preferred_element_type=jnp.float32)
  h = _bn(y + b_ref[...][None, :], gm_ref[...], bt_ref[...])
  z = jnp.dot(h, fc0w_ref[...], preferred_element_type=jnp.float32)
  z = jnp.maximum(z + fc0b_ref[...][None, :], 0.0)
  o = jnp.dot(z, fc1w_ref[...], preferred_element_type=jnp.float32)
  out_ref[...] = o + fc1b_ref[...][None, :]


def kernel(x, edge_index, edge_attr, params):
  n = x.shape[0]
  e = edge_index.shape[1]
  npr = n + PAD_ROWS
  assert npr % NS == 0 and n % NS == 0

  e_f = e + n
  nch = -(-e_f // (NW * CHUNK * NBI)) * NBI
  e_pad = nch * NW * CHUNK
  n_dummy = e_pad - e_f

  src = edge_index[0].astype(jnp.int32)
  dst = edge_index[1].astype(jnp.int32)
  loop = jnp.arange(n, dtype=jnp.int32)
  padidx = n + (jnp.arange(n_dummy, dtype=jnp.int32) % PAD_ROWS)
  src_1d = jnp.concatenate([src, loop, padidx])
  dst_1d = jnp.concatenate([dst, loop, padidx])
  dst_f = dst_1d.reshape(NW, nch, CHUNK)

  deg_pad = -(-(n + PAD_ROWS) // (NS * 128)) * NS * 128
  zeros2d = jnp.zeros((npr, EMB), jnp.float32)
  zeros1d = jnp.zeros((deg_pad,), jnp.float32)
  ones_c = jnp.ones((CHUNK,), jnp.float32)

  edge_k = _edge_kernel(n, PAD_ROWS, e_pad)
  deg_k = _deg_kernel(deg_pad, e_pad)

  degp = deg_k(dst_f, zeros1d, ones_c)

  g, dinv = pl.pallas_call(
      functools.partial(_dense0_body, n=n, npr=npr),
      out_shape=(jax.ShapeDtypeStruct((npr, EMB), jnp.float32),
                 jax.ShapeDtypeStruct((n, 1), jnp.float32)),
  )(x, params['lin_x_W'], params['lin_x_b'], degp)

  for l in range(2):
    sp = edge_k(src_1d, dst_1d, g, zeros2d)
    g = pl.pallas_call(
        functools.partial(_layer_body, n=n, npr=npr),
        out_shape=jax.ShapeDtypeStruct((npr, EMB), jnp.float32),
    )(sp, dinv, params['W%d' % l], params['b%d' % l], params['gamma%d' % l],
      params['beta%d' % l])

  sp = edge_k(src_1d, dst_1d, g, zeros2d)
  out = pl.pallas_call(
      functools.partial(_final_body, n=n),
      out_shape=jax.ShapeDtypeStruct((n, 1), jnp.float32),
  )(sp, dinv, params['W2'], params['b2'], params['gamma2'], params['beta2'],
    params['fc0_W'], params['fc0_b'], params['fc1_W'], params['fc1_b'])
  return out
